# Initial kernel scaffold; baseline (speedup 1.0000x reference)
import jax
import jax.numpy as jnp
from jax.experimental import pallas as pl


def kernel(values, coords, query_coords):
    raise NotImplementedError("write your pallas kernel here")


# speedup vs baseline: 6.2823x; 6.2823x over previous
import jax
import jax.numpy as jnp
from jax.experimental import pallas as pl
from jax.experimental.pallas import tpu as pltpu

KS0_HALF = 0.06
KS1_HALF = 0.06

N_POINTS = 8192
N_QUERIES = 2048
N_CHANNELS = 128

QB = 16


def _dense_kernel(qc_smem, cx_ref, cy_ref, values_ref, out_ref):
    cx = cx_ref[...]
    cy = cy_ref[...]
    for q in range(QB):
        qx = qc_smem[q, 0]
        qy = qc_smem[q, 1]
        mask = (jnp.abs(cx - qx) < KS0_HALF) & (jnp.abs(cy - qy) < KS1_HALF)
        masked = jnp.where(mask, values_ref[...], -jnp.inf)
        out_ref[q, :] = jnp.max(masked, axis=0)


def kernel(values, coords, query_coords):
    cx = coords[:, 0:1]
    cy = coords[:, 1:2]
    grid = (N_QUERIES // QB,)
    out = pl.pallas_call(
        _dense_kernel,
        grid=grid,
        in_specs=[
            pl.BlockSpec((QB, 2), lambda i: (i, 0), memory_space=pltpu.SMEM),
            pl.BlockSpec((N_POINTS, 1), lambda i: (0, 0)),
            pl.BlockSpec((N_POINTS, 1), lambda i: (0, 0)),
            pl.BlockSpec((N_POINTS, N_CHANNELS), lambda i: (0, 0)),
        ],
        out_specs=pl.BlockSpec((QB, N_CHANNELS), lambda i: (i, 0)),
        out_shape=jax.ShapeDtypeStruct((N_QUERIES, N_CHANNELS), jnp.float32),
    )(query_coords, cx, cy, values)
    return out


# speedup vs baseline: 14.3878x; 2.2902x over previous
import functools

import jax
import jax.numpy as jnp
from jax import lax
from jax.experimental import pallas as pl
from jax.experimental.pallas import tpu as pltpu
from jax.experimental.pallas import tpu_sc as plsc

KS0_HALF = 0.06
KS1_HALF = 0.06

N_POINTS = 8192
N_QUERIES = 2048
N_CHANNELS = 128

L = 16
NC = 2
NS = 16
NW = NC * NS
QPW = N_QUERIES // NW
NCHUNK = N_POINTS // L
PROW = N_POINTS // 128
GC = 128
NGMAX = N_POINTS // GC
CB = N_CHANNELS // L

NEG_INF = float("-inf")


def _sc_body(px_hbm, py_hbm, qx_hbm, qy_hbm, values_hbm, out_hbm,
             px_v, py_v, qx_v, qy_v, idx_v, rows_v, out_v, sem):
    cid = lax.axis_index("c")
    sid = lax.axis_index("s")
    wid = sid * NC + cid
    qbase = wid * QPW

    pltpu.sync_copy(px_hbm, px_v)
    pltpu.sync_copy(py_hbm, py_v)
    pltpu.sync_copy(qx_hbm.at[pl.ds(wid * (QPW // L), QPW // L)], qx_v)
    pltpu.sync_copy(qy_hbm.at[pl.ds(wid * (QPW // L), QPW // L)], qy_v)

    lane_iota = lax.iota(jnp.int32, L)

    def zinit(i, _):
        idx_v[pl.ds(i * L, L)] = jnp.zeros((L,), jnp.int32)
        return _
    lax.fori_loop(0, (N_POINTS + GC) // L, zinit, 0)

    def per_query_group(qg, _):
        qxc = qx_v[qg]
        qyc = qy_v[qg]
        for ql in range(L):
            _one_query(qg * L + ql, qxc[ql], qyc[ql])
        return _

    def _one_query(qi, qx, qy):

        def mask_row(jr, count):
            for jc in range(128 // L):
                pxc = px_v[jr, pl.ds(jc * L, L)]
                pyc = py_v[jr, pl.ds(jc * L, L)]
                m = jnp.maximum(jnp.abs(pxc - qx), jnp.abs(pyc - qy)) < KS0_HALF
                nhits = plsc.all_reduce_population_count(m)[0]
                ids = lane_iota + (jr * 128 + jc * L)
                plsc.store_compressed(idx_v.at[pl.ds(count, L)], ids, mask=m)
                count = count + nhits
            return count

        count = lax.fori_loop(0, PROW, mask_row, jnp.int32(0))

        nchunks = (count + (GC - 1)) >> 7

        def gather_chunk(k, accs):
            copy = pltpu.make_async_copy(
                values_hbm.at[idx_v.at[pl.ds(k * GC, GC)]], rows_v, sem)
            copy.start()
            copy.wait()
            rlim = jnp.minimum(jnp.int32(GC), count - k * GC)

            def fold_row(r, accs):
                return tuple(
                    jnp.maximum(accs[cb], rows_v[r, pl.ds(cb * L, L)])
                    for cb in range(CB))

            return lax.fori_loop(0, rlim, fold_row, accs)

        acc0 = tuple(jnp.full((L,), NEG_INF, jnp.float32) for _ in range(CB))
        accs = lax.fori_loop(0, nchunks, gather_chunk, acc0)

        for cb in range(CB):
            out_v[qi, pl.ds(cb * L, L)] = accs[cb]

    lax.fori_loop(0, QPW // L, per_query_group, 0)

    pltpu.sync_copy(out_v, out_hbm.at[pl.ds(qbase, QPW)])


@jax.jit
def _sc_call(px2, py2, qx, qy, values):
    mesh = plsc.VectorSubcoreMesh(core_axis_name="c", subcore_axis_name="s")
    return pl.kernel(
        _sc_body,
        out_type=jax.ShapeDtypeStruct((N_QUERIES, N_CHANNELS), jnp.float32),
        mesh=mesh,
        compiler_params=pltpu.CompilerParams(needs_layout_passes=False),
        scratch_types=[
            pltpu.VMEM((PROW, 128), jnp.float32),
            pltpu.VMEM((PROW, 128), jnp.float32),
            pltpu.VMEM((QPW // L, L), jnp.float32),
            pltpu.VMEM((QPW // L, L), jnp.float32),
            pltpu.VMEM((N_POINTS + GC,), jnp.int32),
            pltpu.VMEM((GC, N_CHANNELS), jnp.float32),
            pltpu.VMEM((QPW, N_CHANNELS), jnp.float32),
            pltpu.SemaphoreType.DMA,
        ],
    )(px2, py2, qx, qy, values)


def kernel(values, coords, query_coords):
    px2 = coords[:, 0].reshape(PROW, 128)
    py2 = coords[:, 1].reshape(PROW, 128)
    qx = query_coords[:, 0].reshape(N_QUERIES // L, L)
    qy = query_coords[:, 1].reshape(N_QUERIES // L, L)
    return _sc_call(px2, py2, qx, qy, values)


# speedup vs baseline: 14.7809x; 1.0273x over previous
import jax
import jax.numpy as jnp
from jax import lax
from jax.experimental import pallas as pl
from jax.experimental.pallas import tpu as pltpu
from jax.experimental.pallas import tpu_sc as plsc

KS_HALF = 0.06

N_POINTS = 8192
N_QUERIES = 2048
N_CHANNELS = 128

L = 16
NC = 2
NS = 16
NW = NC * NS
QPW = N_QUERIES // NW
PROW = N_POINTS // 128
NB = 128
GC = 128
CB = N_CHANNELS // L

NEG_INF = float("-inf")


def _sc_body(px_hbm, py_hbm, qx_hbm, qy_hbm, values_hbm, out_hbm,
             px_v, py_v, qx_v, qy_v, px_s, py_s, ids_s,
             counts_v, offs_v, curs_v, idx_v, rows_v, out_v, sem):
    cid = lax.axis_index("c")
    sid = lax.axis_index("s")
    wid = sid * NC + cid
    qbase = wid * QPW

    pltpu.sync_copy(px_hbm, px_v)
    pltpu.sync_copy(py_hbm, py_v)
    pltpu.sync_copy(qx_hbm.at[pl.ds(wid * (QPW // L), QPW // L)], qx_v)
    pltpu.sync_copy(qy_hbm.at[pl.ds(wid * (QPW // L), QPW // L)], qy_v)

    lane_iota = lax.iota(jnp.int32, L)
    zeros_i = jnp.zeros((L,), jnp.int32)

    def zinit(i, _):
        idx_v[pl.ds(i * L, L)] = zeros_i
        return _

    lax.fori_loop(0, (N_POINTS + GC) // L, zinit, 0)

    def zinit_ids(i, _):
        ids_s[pl.ds(i * L, L)] = zeros_i
        return _

    lax.fori_loop(0, N_POINTS // L, zinit_ids, 0)

    dcbase = plsc.scan_count(zeros_i)[0][0]

    for c in range(NB // L):
        counts_v[c] = zeros_i

    def hist_row(jr, _):
        for jc in range(128 // L):
            pxc = px_v[jr, pl.ds(jc * L, L)]
            b = (pxc * NB).astype(jnp.int32)
            dc, last = plsc.scan_count(b)
            plsc.addupdate_scatter(
                counts_v, [b >> 4, b & (L - 1)], dc + 1 - dcbase, mask=last)
        return _

    lax.fori_loop(0, PROW, hist_row, 0)

    run = jnp.int32(0)
    for c in range(NB // L):
        cnt = counts_v[c]
        cs = plsc.cumsum(cnt)
        excl = cs - cnt + run
        offs_v[c] = excl
        curs_v[c] = excl
        run = run + cs[L - 1]

    def sort_row(jr, _):
        for jc in range(128 // L):
            pxc = px_v[jr, pl.ds(jc * L, L)]
            pyc = py_v[jr, pl.ds(jc * L, L)]
            b = (pxc * NB).astype(jnp.int32)
            dc, last = plsc.scan_count(b)
            base = plsc.load_gather(curs_v, [b >> 4, b & (L - 1)])
            pos = jnp.clip(base + dc - dcbase, 0, N_POINTS - 1)
            ids = lane_iota + (jr * 128 + jc * L)
            plsc.store_scatter(px_s, [pos], pxc)
            plsc.store_scatter(py_s, [pos], pyc)
            plsc.store_scatter(ids_s, [pos], ids)
            plsc.addupdate_scatter(
                curs_v, [b >> 4, b & (L - 1)], dc + 1 - dcbase, mask=last)
        return _

    lax.fori_loop(0, PROW, sort_row, 0)

    def _one_query(qi, qx, qy, lo, hi):
        def mask_chunk(jj, count):
            o = jj * L
            pxc = px_s[pl.ds(o, L)]
            pyc = py_s[pl.ds(o, L)]
            m = jnp.maximum(jnp.abs(pxc - qx), jnp.abs(pyc - qy)) < KS_HALF
            nhits = plsc.all_reduce_population_count(m)[0]
            ids = ids_s[pl.ds(o, L)]
            plsc.store_compressed(idx_v.at[pl.ds(count, L)], ids, mask=m)
            return count + nhits

        count = lax.fori_loop(lo >> 4, (hi + L - 1) >> 4, mask_chunk,
                              jnp.int32(0))

        nchunks = (count + (GC - 1)) >> 7

        def gather_chunk(k, accs):
            copy = pltpu.make_async_copy(
                values_hbm.at[idx_v.at[pl.ds(k * GC, GC)]], rows_v, sem)
            copy.start()
            copy.wait()
            rlim = jnp.minimum(jnp.int32(GC), count - k * GC)

            def fold_row(r, accs):
                return tuple(
                    jnp.maximum(accs[cb], rows_v[r, pl.ds(cb * L, L)])
                    for cb in range(CB))

            return lax.fori_loop(0, rlim, fold_row, accs)

        acc0 = tuple(jnp.full((L,), NEG_INF, jnp.float32) for _ in range(CB))
        accs = lax.fori_loop(0, nchunks, gather_chunk, acc0)

        for cb in range(CB):
            out_v[qi, pl.ds(cb * L, L)] = accs[cb]

    def per_query_group(qg, _):
        qxc = qx_v[qg]
        qyc = qy_v[qg]
        blo = jnp.maximum(((qxc - KS_HALF) * NB).astype(jnp.int32), 0)
        bhi = jnp.minimum(((qxc + KS_HALF) * NB).astype(jnp.int32), NB - 1)
        lo_vec = jnp.clip(
            plsc.load_gather(offs_v, [blo >> 4, blo & (L - 1)]), 0, N_POINTS)
        hi_vec = jnp.clip(
            plsc.load_gather(curs_v, [bhi >> 4, bhi & (L - 1)]), 0, N_POINTS)
        for ql in range(L):
            _one_query(qg * L + ql, qxc[ql], qyc[ql], lo_vec[ql], hi_vec[ql])
        return _

    lax.fori_loop(0, QPW // L, per_query_group, 0)

    pltpu.sync_copy(out_v, out_hbm.at[pl.ds(qbase, QPW)])


@jax.jit
def _sc_call(px2, py2, qx, qy, values):
    mesh = plsc.VectorSubcoreMesh(core_axis_name="c", subcore_axis_name="s")
    return pl.kernel(
        _sc_body,
        out_type=jax.ShapeDtypeStruct((N_QUERIES, N_CHANNELS), jnp.float32),
        mesh=mesh,
        compiler_params=pltpu.CompilerParams(needs_layout_passes=False),
        scratch_types=[
            pltpu.VMEM((PROW, 128), jnp.float32),
            pltpu.VMEM((PROW, 128), jnp.float32),
            pltpu.VMEM((QPW // L, L), jnp.float32),
            pltpu.VMEM((QPW // L, L), jnp.float32),
            pltpu.VMEM((N_POINTS,), jnp.float32),
            pltpu.VMEM((N_POINTS,), jnp.float32),
            pltpu.VMEM((N_POINTS,), jnp.int32),
            pltpu.VMEM((NB // L, L), jnp.int32),
            pltpu.VMEM((NB // L, L), jnp.int32),
            pltpu.VMEM((NB // L, L), jnp.int32),
            pltpu.VMEM((N_POINTS + GC,), jnp.int32),
            pltpu.VMEM((GC, N_CHANNELS), jnp.float32),
            pltpu.VMEM((QPW, N_CHANNELS), jnp.float32),
            pltpu.SemaphoreType.DMA,
        ],
    )(px2, py2, qx, qy, values)


def kernel(values, coords, query_coords):
    px2 = coords[:, 0].reshape(PROW, 128)
    py2 = coords[:, 1].reshape(PROW, 128)
    qx = query_coords[:, 0].reshape(N_QUERIES // L, L)
    qy = query_coords[:, 1].reshape(N_QUERIES // L, L)
    return _sc_call(px2, py2, qx, qy, values)
